# baseline (speedup 1.0000x reference)
import functools
import jax
import jax.numpy as jnp
from jax import lax
from jax.experimental import pallas as pl
from jax.experimental.pallas import tpu as pltpu
from jax.experimental.pallas import tpu_sc as plsc

DIM = 64
TOTAL = 4096 * 200
NC, NS = 2, 16
NW = NC * NS
PER_W = TOTAL // NW
CHUNK = 1024
SUB = 128
N_SUB = CHUNK // SUB
N_OUTER = PER_W // CHUNK


@functools.partial(
    pl.kernel,
    out_type=jax.ShapeDtypeStruct((TOTAL, DIM), jnp.float32),
    mesh=plsc.VectorSubcoreMesh(core_axis_name="c", subcore_axis_name="s"),
    scratch_types=[
        pltpu.VMEM((CHUNK,), jnp.int32),
        pltpu.VMEM((CHUNK, DIM), jnp.float32),
        pltpu.SemaphoreType.DMA,
    ],
    compiler_params=pltpu.CompilerParams(use_tc_tiling_on_sc=False),
)
def _emb_lookup(idx_hbm, table_hbm, out_hbm, idx_v, rows_v, sem):
    wid = lax.axis_index("s") * NC + lax.axis_index("c")
    base = wid * PER_W

    @pl.loop(0, N_OUTER)
    def _outer(i):
        start = base + i * CHUNK
        pltpu.sync_copy(idx_hbm.at[pl.ds(start, CHUNK)], idx_v)
        copies = [
            pltpu.async_copy(
                table_hbm.at[idx_v.at[pl.ds(j * SUB, SUB)]],
                rows_v.at[pl.ds(j * SUB, SUB)],
                sem,
            )
            for j in range(N_SUB)
        ]
        for c in copies:
            c.wait()
        pltpu.sync_copy(rows_v, out_hbm.at[pl.ds(start, CHUNK)])


def kernel(x, embedding):
    flat = x.reshape(TOTAL)
    out = _emb_lookup(flat, embedding)
    return out.reshape(x.shape + (DIM,))


# speedup vs baseline: 1.0182x; 1.0182x over previous
import functools
import jax
import jax.numpy as jnp
from jax import lax
from jax.experimental import pallas as pl
from jax.experimental.pallas import tpu as pltpu
from jax.experimental.pallas import tpu_sc as plsc

DIM = 64
TOTAL = 4096 * 200
NC, NS = 2, 16
NW = NC * NS
PER_W = TOTAL // NW
CHUNK = 640
SUB = 128
N_SUB = CHUNK // SUB
N_CHUNK = PER_W // CHUNK
N_RBUF = 2
N_IBUF = 4


@functools.partial(
    pl.kernel,
    out_type=jax.ShapeDtypeStruct((TOTAL, DIM), jnp.float32),
    mesh=plsc.VectorSubcoreMesh(core_axis_name="c", subcore_axis_name="s"),
    scratch_types=[
        pltpu.VMEM((N_IBUF, CHUNK), jnp.int32),
        pltpu.VMEM((N_RBUF, CHUNK, DIM), jnp.float32),
        pltpu.SemaphoreType.DMA,
        pltpu.SemaphoreType.DMA,
        pltpu.SemaphoreType.DMA,
    ],
    compiler_params=pltpu.CompilerParams(use_tc_tiling_on_sc=False),
)
def _emb_lookup(idx_hbm, table_hbm, out_hbm, idx_v, rows_v, idx_sem, gat_sem, out_sem):
    wid = lax.axis_index("s") * NC + lax.axis_index("c")
    base = wid * PER_W

    def idx_copy(c, ib):
        return pltpu.make_async_copy(
            idx_hbm.at[pl.ds(base + c * CHUNK, CHUNK)], idx_v.at[ib], idx_sem)

    def gathers(ib, rb):
        return [
            pltpu.make_async_copy(
                table_hbm.at[idx_v.at[ib, pl.ds(j * SUB, SUB)]],
                rows_v.at[rb, pl.ds(j * SUB, SUB)],
                gat_sem)
            for j in range(N_SUB)
        ]

    def out_copy(c, rb):
        return pltpu.make_async_copy(
            rows_v.at[rb], out_hbm.at[pl.ds(base + c * CHUNK, CHUNK)], out_sem)

    idx_copy(0, 0).start()
    idx_copy(1, 1).start()

    @pl.loop(0, N_CHUNK, step=N_IBUF)
    def _g(g):
        for u in range(N_IBUF):
            c = g + u
            rb = u % N_RBUF
            idx_copy(c, u).wait()

            @pl.when(c >= N_RBUF)
            def _():
                out_copy(c - N_RBUF, rb).wait()

            for d in gathers(u, rb):
                d.start()

            @pl.when(c + 2 < N_CHUNK)
            def _():
                idx_copy(c + 2, (u + 2) % N_IBUF).start()

            @pl.when(c >= 1)
            def _():
                for d in gathers((u + N_IBUF - 1) % N_IBUF, 1 - rb):
                    d.wait()
                out_copy(c - 1, 1 - rb).start()

    last_rb = (N_CHUNK - 1) % N_RBUF
    for d in gathers((N_CHUNK - 1) % N_IBUF, last_rb):
        d.wait()
    out_copy(N_CHUNK - 1, last_rb).start()
    out_copy(N_CHUNK - 2, 1 - last_rb).wait()
    out_copy(N_CHUNK - 1, last_rb).wait()


def kernel(x, embedding):
    flat = x.reshape(TOTAL)
    out = _emb_lookup(flat, embedding)
    return out.reshape(x.shape + (DIM,))
